# baseline (speedup 1.0000x reference)
import functools

import numpy as np
import jax
import jax.numpy as jnp
from jax import lax
from jax.experimental import pallas as pl
from jax.experimental.pallas import tpu as pltpu
from jax.experimental.pallas import tpu_sc as plsc

B = 64
N = 8192
D_WHAT = 64
MAXO = 10
N_OBJ = MAXO + int(0.1 * N)
PAD = 832
CH = 128
NCH = 7
SPAN = NCH * CH
EPS = 1e-3

_RANK_CACHE = None


def _rotl32(x, r):
    return ((x << np.uint32(r)) | (x >> np.uint32(32 - r))).astype(np.uint32)


def _threefry2x32(k0, k1, x0, x1):
    rotations = [[13, 15, 26, 6], [17, 29, 16, 24]]
    ks = [np.uint32(k0), np.uint32(k1),
          np.uint32(k0) ^ np.uint32(k1) ^ np.uint32(0x1BD11BDA)]
    x = [x0.astype(np.uint32) + ks[0], x1.astype(np.uint32) + ks[1]]
    for i in range(5):
        for r in rotations[i % 2]:
            x[0] = (x[0] + x[1]).astype(np.uint32)
            x[1] = _rotl32(x[1], r)
            x[1] = x[1] ^ x[0]
        x[0] = (x[0] + ks[(i + 1) % 3]).astype(np.uint32)
        x[1] = (x[1] + ks[(i + 2) % 3] + np.uint32(i + 1)).astype(np.uint32)
    return x[0], x[1]


def _score_rank():
    global _RANK_CACHE
    if _RANK_CACHE is None:
        total = B * N
        o0, o1 = _threefry2x32(
            np.uint32(0), np.uint32(12345),
            np.zeros(total, np.uint32), np.arange(total, dtype=np.uint32))
        bits = o0 ^ o1
        fl = ((bits >> np.uint32(9)) | np.uint32(0x3F800000)).view(np.float32)
        score = np.maximum(np.float32(0.0),
                           fl - np.float32(1.0)).reshape(B, N)
        order = np.argsort(-score, axis=1, kind="stable")
        rank = np.argsort(order, axis=1, kind="stable")
        _RANK_CACHE = rank.astype(np.int32)
    return _RANK_CACHE


def _tc_body(zp_ref, rank_ref, tag_ref, pos_ref):
    v = zp_ref[...]
    r = rank_ref[...]
    present = jnp.where(v > 0.5, 1, 0).astype(jnp.int32)
    iota = lax.broadcasted_iota(jnp.int32, (B, N), 1)

    def step(_, carry):
        active, kept, c = carry
        af = active.astype(jnp.float32)
        masked_v = v * af + (af - 1.0)
        m = jnp.max(masked_v, axis=1, keepdims=True)
        cand = active * jnp.where(v == m, 1, 0)
        selidx = jnp.max(cand * iota + (cand - 1), axis=1, keepdims=True)
        chosen = active * jnp.where(iota == selidx, 1, 0)
        rj = jnp.max(chosen * r + (chosen - 1), axis=1, keepdims=True)
        validc = jnp.where(selidx >= 0, 1, 0)
        c = c + validc * jnp.where(rj < r, 1, 0)
        kept = jnp.maximum(kept, chosen)
        active = active * (1 - chosen)
        return active, kept, c

    carry0 = (present, jnp.zeros((B, N), jnp.int32), jnp.zeros((B, N), jnp.int32))
    _, kept, c = lax.fori_loop(0, MAXO, step, carry0)
    n2 = jnp.sum(kept, axis=1, keepdims=True)
    negneed = N_OBJ - n2
    negative = (1 - kept) * jnp.where((r - c) < negneed, 1, 0)
    keep = kept + negative
    tag = (kept.astype(jnp.float32)
           - negative.astype(jnp.float32) * jnp.where(v > EPS, 1.0, 2.0))
    x = keep
    s = 1
    while s < N:
        x = x + jnp.concatenate(
            [jnp.zeros((B, s), jnp.int32), x[:, :N - s]], axis=1)
        s *= 2
    pos = x - 1
    tag_ref[...] = tag
    pos_ref[...] = pos


def _phase_a(zp2d, rank):
    return pl.pallas_call(
        _tc_body,
        out_shape=[
            jax.ShapeDtypeStruct((B, N), jnp.float32),
            jax.ShapeDtypeStruct((B, N), jnp.int32),
        ],
    )(zp2d, rank)


def _sc_body(tag_hbm, pos_hbm, where_hbm, wloc_hbm, wscale_hbm, dloc_hbm,
             dscale_hbm,
             owhat_hbm, owhere_hbm, opres_hbm, odepth_hbm,
             tag_v, pos_v, idx_v, gidx_v, ptag_v, wrow_v, dloc_v, dsc_v,
             loc_c, sc_c, owhat_c, owhere_v, opres_v, odepth_v,
             sem, sem2):
    wid = lax.axis_index("s") * 2 + lax.axis_index("c")

    def do_row(b):
        pltpu.sync_copy(tag_hbm.at[b], tag_v)
        pltpu.sync_copy(pos_hbm.at[b], pos_v)
        pltpu.sync_copy(where_hbm.at[b], wrow_v)
        pltpu.sync_copy(dloc_hbm.at[b], dloc_v)
        pltpu.sync_copy(dscale_hbm.at[b], dsc_v)
        for k in range((SPAN - 816) // 16):
            idx_v[pl.ds(816 + 16 * k, 16)] = jnp.zeros((16,), jnp.int32)

        def comp(g, _):
            r, o = lax.shift_right_logical(g, 3), (g & 7) * 16
            t16 = tag_v[r, pl.ds(o, 16)]
            keep16 = t16 != 0.0
            p16 = pos_v[r, pl.ds(o, 16)]
            cols = lax.iota(jnp.int32, 16) + g * 16
            plsc.store_scatter(idx_v, [p16], cols, mask=keep16)
            plsc.store_scatter(ptag_v, [p16], t16, mask=keep16)
            return 0

        lax.fori_loop(0, N // 16, comp, 0)

        def small(g, _):
            c16 = idx_v[pl.ds(g * 16, 16)]
            gidx_v[pl.ds(g * 16, 16)] = b * (N // 2) + lax.shift_right_logical(c16, 1)
            t16 = ptag_v[pl.ds(g * 16, 16)]
            eps16 = t16 > -1.5
            r, o = lax.shift_right_logical(g, 3), (g & 7) * 16
            opres_v[r, pl.ds(o, 16)] = jnp.where(t16 > 0.0, 1.0, -1.0)
            dl = plsc.load_gather(
                dloc_v, [lax.shift_right_logical(c16, 7), c16 & 127])
            dsv = plsc.load_gather(
                dsc_v, [lax.shift_right_logical(c16, 7), c16 & 127])
            odepth_v[r, pl.ds(o, 16)] = jnp.where(eps16, dl + dsv, 1.0)
            outp = (lax.iota(jnp.int32, 16) + g * 16) * 4
            for d in range(4):
                f16 = c16 * 4 + d
                wv = plsc.load_gather(
                    wrow_v, [lax.shift_right_logical(f16, 7), f16 & 127])
                q16 = outp + d
                plsc.store_scatter(
                    owhere_v, [lax.shift_right_logical(q16, 7), q16 & 127], wv)
            return 0

        lax.fori_loop(0, SPAN // 16, small, 0)

        for c in range(NCH):
            cp = pltpu.async_copy(
                wloc_hbm.at[gidx_v.at[pl.ds(c * CH, CH)]], loc_c, sem)
            cp2 = pltpu.async_copy(
                wscale_hbm.at[gidx_v.at[pl.ds(c * CH, CH)]], sc_c, sem2)
            cp.wait()
            cp2.wait()

            def wcomp(p, _):
                splat = jnp.full((16,), c * CH, jnp.int32) + p
                tsp = plsc.load_gather(ptag_v, [splat])
                eps16 = tsp > -1.5
                csp = plsc.load_gather(idx_v, [splat])
                half = (jnp.max(csp) & 1) * D_WHAT
                orow = lax.shift_right_logical(p, 1)
                obase = (p & 1) * D_WHAT
                for dc in range(D_WHAT // 16):
                    l16 = loc_c[p, pl.ds(half + dc * 16, 16)]
                    s16 = sc_c[p, pl.ds(half + dc * 16, 16)]
                    owhat_c[orow, pl.ds(obase + dc * 16, 16)] = jnp.where(
                        eps16, l16 + s16, 1.0)
                return 0

            lax.fori_loop(0, CH, wcomp, 0)
            pltpu.sync_copy(owhat_c,
                            owhat_hbm.at[b, pl.ds(c * (CH // 2), CH // 2)])

        pltpu.sync_copy(opres_v, opres_hbm.at[b])
        pltpu.sync_copy(odepth_v, odepth_hbm.at[b])
        pltpu.sync_copy(owhere_v, owhere_hbm.at[b])

    for rr in range(2):
        do_row(wid * 2 + rr)


def _phase_b(tag, pos, where_flat, wloc, wscale, dloc2, dscale2):
    mesh = plsc.VectorSubcoreMesh(core_axis_name="c", subcore_axis_name="s")
    fn = functools.partial(
        pl.kernel,
        mesh=mesh,
        compiler_params=pltpu.CompilerParams(needs_layout_passes=False),
        out_type=[
            jax.ShapeDtypeStruct((B, SPAN * D_WHAT // 128, 128), jnp.float32),
            jax.ShapeDtypeStruct((B, 4 * SPAN // 128, 128), jnp.float32),
            jax.ShapeDtypeStruct((B, 8, 128), jnp.float32),
            jax.ShapeDtypeStruct((B, 8, 128), jnp.float32),
        ],
        scratch_types=[
            pltpu.VMEM((N // 128, 128), jnp.float32),
            pltpu.VMEM((N // 128, 128), jnp.int32),
            pltpu.VMEM((SPAN,), jnp.int32),
            pltpu.VMEM((SPAN,), jnp.int32),
            pltpu.VMEM((SPAN,), jnp.float32),
            pltpu.VMEM((4 * N // 128, 128), jnp.float32),
            pltpu.VMEM((N // 128, 128), jnp.float32),
            pltpu.VMEM((N // 128, 128), jnp.float32),
            pltpu.VMEM((CH, 128), jnp.float32),
            pltpu.VMEM((CH, 128), jnp.float32),
            pltpu.VMEM((CH // 2, 128), jnp.float32),
            pltpu.VMEM((4 * SPAN // 128, 128), jnp.float32),
            pltpu.VMEM((8, 128), jnp.float32),
            pltpu.VMEM((8, 128), jnp.float32),
            pltpu.SemaphoreType.DMA,
            pltpu.SemaphoreType.DMA,
        ],
    )(_sc_body)
    return fn(tag, pos, where_flat, wloc, wscale, dloc2, dscale2)


def kernel(z_where, z_present, z_what_loc, z_what_scale,
           z_depth_loc, z_depth_scale):
    rank = jnp.asarray(_score_rank())
    zp = z_present[:, :, 0]
    tag, pos = _phase_a(zp, rank)
    owhat, owhere, opres, odepth = _phase_b(
        tag.reshape(B, N // 128, 128),
        pos.reshape(B, N // 128, 128),
        z_where.reshape(B, 4 * N // 128, 128),
        z_what_loc.reshape(B * N // 2, 2 * D_WHAT),
        z_what_scale.reshape(B * N // 2, 2 * D_WHAT),
        z_depth_loc.reshape(B, N // 128, 128),
        z_depth_scale.reshape(B, N // 128, 128),
    )
    out_where = owhere.reshape(B, SPAN, 4)[:, :N_OBJ, :]
    out_pres = opres.reshape(B, 1024)[:, :N_OBJ, None]
    out_depth = odepth.reshape(B, 1024)[:, :N_OBJ, None]
    out_what = owhat.reshape(B, SPAN, D_WHAT)[:, :N_OBJ, :]
    return (out_where, out_pres, out_what, out_depth)


# speedup vs baseline: 1.5407x; 1.5407x over previous
import functools

import numpy as np
import jax
import jax.numpy as jnp
from jax import lax
from jax.experimental import pallas as pl
from jax.experimental.pallas import tpu as pltpu
from jax.experimental.pallas import tpu_sc as plsc

B = 64
N = 8192
D_WHAT = 64
MAXO = 10
N_OBJ = MAXO + int(0.1 * N)
PAD = 832
OPAD = 896
NB = 128
NBLK = N // NB
EPS = 1e-3

_RANK_CACHE = None


def _rotl32(x, r):
    return ((x << np.uint32(r)) | (x >> np.uint32(32 - r))).astype(np.uint32)


def _threefry2x32(k0, k1, x0, x1):
    rotations = [[13, 15, 26, 6], [17, 29, 16, 24]]
    ks = [np.uint32(k0), np.uint32(k1),
          np.uint32(k0) ^ np.uint32(k1) ^ np.uint32(0x1BD11BDA)]
    x = [x0.astype(np.uint32) + ks[0], x1.astype(np.uint32) + ks[1]]
    for i in range(5):
        for r in rotations[i % 2]:
            x[0] = (x[0] + x[1]).astype(np.uint32)
            x[1] = _rotl32(x[1], r)
            x[1] = x[1] ^ x[0]
        x[0] = (x[0] + ks[(i + 1) % 3]).astype(np.uint32)
        x[1] = (x[1] + ks[(i + 2) % 3] + np.uint32(i + 1)).astype(np.uint32)
    return x[0], x[1]


def _score_rank():
    global _RANK_CACHE
    if _RANK_CACHE is None:
        total = B * N
        o0, o1 = _threefry2x32(
            np.uint32(0), np.uint32(12345),
            np.zeros(total, np.uint32), np.arange(total, dtype=np.uint32))
        bits = o0 ^ o1
        fl = ((bits >> np.uint32(9)) | np.uint32(0x3F800000)).view(np.float32)
        score = np.maximum(np.float32(0.0),
                           fl - np.float32(1.0)).reshape(B, N)
        order = np.argsort(-score, axis=1, kind="stable")
        rank = np.argsort(order, axis=1, kind="stable")
        _RANK_CACHE = rank.astype(np.int32)
    return _RANK_CACHE


def _tc_body(zp_ref, rank_ref, tag_ref, pos_ref):
    v = zp_ref[...]
    r = rank_ref[...]
    present = jnp.where(v > 0.5, 1, 0).astype(jnp.int32)
    iota = lax.broadcasted_iota(jnp.int32, (B, N), 1)

    def step(_, carry):
        active, kept, c = carry
        af = active.astype(jnp.float32)
        masked_v = v * af + (af - 1.0)
        m = jnp.max(masked_v, axis=1, keepdims=True)
        cand = active * jnp.where(v == m, 1, 0)
        selidx = jnp.max(cand * iota + (cand - 1), axis=1, keepdims=True)
        chosen = active * jnp.where(iota == selidx, 1, 0)
        rj = jnp.max(chosen * r + (chosen - 1), axis=1, keepdims=True)
        validc = jnp.where(selidx >= 0, 1, 0)
        c = c + validc * jnp.where(rj < r, 1, 0)
        kept = jnp.maximum(kept, chosen)
        active = active * (1 - chosen)
        return active, kept, c

    carry0 = (present, jnp.zeros((B, N), jnp.int32), jnp.zeros((B, N), jnp.int32))
    _, kept, c = lax.fori_loop(0, MAXO, step, carry0)
    n2 = jnp.sum(kept, axis=1, keepdims=True)
    negneed = N_OBJ - n2
    negative = (1 - kept) * jnp.where((r - c) < negneed, 1, 0)
    keep = kept + negative
    tag = (kept.astype(jnp.float32)
           - negative.astype(jnp.float32) * jnp.where(v > EPS, 1.0, 2.0))
    x = keep
    s = 1
    while s < N:
        x = x + jnp.concatenate(
            [jnp.zeros((B, s), jnp.int32), x[:, :N - s]], axis=1)
        s *= 2
    pos = x - 1
    tag_ref[...] = tag
    pos_ref[...] = pos


def _phase_a(zp2d, rank):
    return pl.pallas_call(
        _tc_body,
        out_shape=[
            jax.ShapeDtypeStruct((B, N), jnp.float32),
            jax.ShapeDtypeStruct((B, N), jnp.int32),
        ],
    )(zp2d, rank)


_SC_PARAMS = pltpu.CompilerParams(needs_layout_passes=False)


def _b1_body(tag_hbm, pos_hbm, where_hbm, dloc_hbm, dscale_hbm,
             oidx_hbm, optag_hbm, owhere_hbm, opres_hbm, odepth_hbm,
             tag_v, pos_v, idx_v, ptag_v, wrow_v, dloc_v, dsc_v,
             owhere_v, opres_v, odepth_v):
    wid = lax.axis_index("s") * 2 + lax.axis_index("c")

    def do_row(b):
        pltpu.sync_copy(tag_hbm.at[b], tag_v)
        pltpu.sync_copy(pos_hbm.at[b], pos_v)
        pltpu.sync_copy(where_hbm.at[b], wrow_v)
        pltpu.sync_copy(dloc_hbm.at[b], dloc_v)
        pltpu.sync_copy(dscale_hbm.at[b], dsc_v)

        def fill(g, _):
            idx_v[lax.shift_right_logical(g, 3), pl.ds((g & 7) * 16, 16)] = (
                jnp.full((16,), N - 1, jnp.int32))
            return 0

        lax.fori_loop(0, 1024 // 16, fill, 0)

        def comp(g, _):
            r, o = lax.shift_right_logical(g, 3), (g & 7) * 16
            t16 = tag_v[r, pl.ds(o, 16)]
            keep16 = t16 != 0.0
            p16 = pos_v[r, pl.ds(o, 16)]
            cols = lax.iota(jnp.int32, 16) + g * 16
            pr, pc = lax.shift_right_logical(p16, 7), p16 & 127
            plsc.store_scatter(idx_v, [pr, pc], cols, mask=keep16)
            plsc.store_scatter(ptag_v, [pr, pc], t16, mask=keep16)
            return 0

        lax.fori_loop(0, N // 16, comp, 0)

        def small(g, _):
            r, o = lax.shift_right_logical(g, 3), (g & 7) * 16
            c16 = idx_v[r, pl.ds(o, 16)]
            t16 = ptag_v[r, pl.ds(o, 16)]
            eps16 = t16 > -1.5
            opres_v[r, pl.ds(o, 16)] = jnp.where(t16 > 0.0, 1.0, -1.0)
            dl = plsc.load_gather(
                dloc_v, [lax.shift_right_logical(c16, 7), c16 & 127])
            dsv = plsc.load_gather(
                dsc_v, [lax.shift_right_logical(c16, 7), c16 & 127])
            odepth_v[r, pl.ds(o, 16)] = jnp.where(eps16, dl + dsv, 1.0)
            outp = (lax.iota(jnp.int32, 16) + g * 16) * 4
            for d in range(4):
                f16 = c16 * 4 + d
                wv = plsc.load_gather(
                    wrow_v, [lax.shift_right_logical(f16, 7), f16 & 127])
                q16 = outp + d
                plsc.store_scatter(
                    owhere_v, [lax.shift_right_logical(q16, 7), q16 & 127], wv)
            return 0

        lax.fori_loop(0, PAD // 16, small, 0)

        pltpu.sync_copy(idx_v, oidx_hbm.at[b])
        pltpu.sync_copy(ptag_v, optag_hbm.at[b])
        pltpu.sync_copy(opres_v, opres_hbm.at[b])
        pltpu.sync_copy(odepth_v, odepth_hbm.at[b])
        pltpu.sync_copy(owhere_v, owhere_hbm.at[b])

    for rr in range(2):
        do_row(wid * 2 + rr)


def _phase_b1(tag, pos, where_flat, dloc2, dscale2):
    mesh = plsc.VectorSubcoreMesh(core_axis_name="c", subcore_axis_name="s")
    fn = functools.partial(
        pl.kernel,
        mesh=mesh,
        compiler_params=_SC_PARAMS,
        out_type=[
            jax.ShapeDtypeStruct((B, 8, 128), jnp.int32),
            jax.ShapeDtypeStruct((B, 8, 128), jnp.float32),
            jax.ShapeDtypeStruct((B, 4 * OPAD // 128, 128), jnp.float32),
            jax.ShapeDtypeStruct((B, 8, 128), jnp.float32),
            jax.ShapeDtypeStruct((B, 8, 128), jnp.float32),
        ],
        scratch_types=[
            pltpu.VMEM((N // 128, 128), jnp.float32),
            pltpu.VMEM((N // 128, 128), jnp.int32),
            pltpu.VMEM((8, 128), jnp.int32),
            pltpu.VMEM((8, 128), jnp.float32),
            pltpu.VMEM((4 * N // 128, 128), jnp.float32),
            pltpu.VMEM((N // 128, 128), jnp.float32),
            pltpu.VMEM((N // 128, 128), jnp.float32),
            pltpu.VMEM((4 * OPAD // 128, 128), jnp.float32),
            pltpu.VMEM((8, 128), jnp.float32),
            pltpu.VMEM((8, 128), jnp.float32),
        ],
    )(_b1_body)
    return fn(tag, pos, where_flat, dloc2, dscale2)


def _b2_body(loc_hbm, sc_hbm, idx_hbm, ptag_hbm, owhat_hbm,
             idx_v, ptag_v,
             loc0, loc1, sc0, sc1, owhat_t,
             seml0, seml1, sems0, sems1):
    wid = lax.axis_index("s") * 2 + lax.axis_index("c")
    iota16 = lax.iota(jnp.int32, 16)

    def _col_at(p):
        sp = jnp.zeros((16,), jnp.int32) + p
        return jnp.max(plsc.load_gather(
            idx_v, [lax.shift_right_logical(sp, 7), sp & 127]))

    def do_row(b):
        pltpu.sync_copy(idx_hbm.at[b], idx_v)
        pltpu.sync_copy(ptag_hbm.at[b], ptag_v)

        locb = (loc0, loc1)
        scb = (sc0, sc1)
        seml = (seml0, seml1)
        sems = (sems0, sems1)

        def start(nb, j):
            off = pl.multiple_of(nb * NB, NB)
            return (
                pltpu.async_copy(loc_hbm.at[b, :, pl.ds(off, NB)],
                                 locb[j], seml[j]),
                pltpu.async_copy(sc_hbm.at[b, :, pl.ds(off, NB)],
                                 scb[j], sems[j]),
            )

        def process(nb, j, ps):
            end = nb * NB + NB
            pe = lax.while_loop(
                lambda p: jnp.logical_and(p < PAD, _col_at(p) < end),
                lambda p: p + 1, ps)
            lb, sb = locb[j], scb[j]

            def body(p, _):
                sp = jnp.zeros((16,), jnp.int32) + p
                spr, spc = lax.shift_right_logical(sp, 7), sp & 127
                c = jnp.max(plsc.load_gather(idx_v, [spr, spc]))
                coff = jnp.zeros((16,), jnp.int32) + (c - nb * NB)
                tsp = plsc.load_gather(ptag_v, [spr, spc])
                eps16 = tsp > -1.5
                for dc in range(D_WHAT // 16):
                    rows = iota16 + dc * 16
                    lg = plsc.load_gather(lb, [rows, coff])
                    sg = plsc.load_gather(sb, [rows, coff])
                    v = jnp.where(eps16, lg + sg, 1.0)
                    plsc.store_scatter(owhat_t, [rows, sp], v)
                return 0

            lax.fori_loop(ps, pe, body, 0)
            return pe

        def pair(k, ps):
            nb0 = k * 2
            h0 = start(nb0, 0)
            h1 = start(nb0 + 1, 1)
            h0[0].wait()
            h0[1].wait()
            ps = process(nb0, 0, ps)
            h1[0].wait()
            h1[1].wait()
            ps = process(nb0 + 1, 1, ps)
            return ps

        lax.fori_loop(0, NBLK // 2, pair, jnp.int32(0))
        pltpu.sync_copy(owhat_t, owhat_hbm.at[b])

    for rr in range(2):
        do_row(wid * 2 + rr)


def _phase_b2(loc_t, scale_t, idx, ptag):
    mesh = plsc.VectorSubcoreMesh(core_axis_name="c", subcore_axis_name="s")
    fn = functools.partial(
        pl.kernel,
        mesh=mesh,
        compiler_params=_SC_PARAMS,
        out_type=[
            jax.ShapeDtypeStruct((B, D_WHAT, OPAD), jnp.float32),
        ],
        scratch_types=[
            pltpu.VMEM((8, 128), jnp.int32),
            pltpu.VMEM((8, 128), jnp.float32),
            pltpu.VMEM((D_WHAT, NB), jnp.float32),
            pltpu.VMEM((D_WHAT, NB), jnp.float32),
            pltpu.VMEM((D_WHAT, NB), jnp.float32),
            pltpu.VMEM((D_WHAT, NB), jnp.float32),
            pltpu.VMEM((D_WHAT, OPAD), jnp.float32),
            pltpu.SemaphoreType.DMA,
            pltpu.SemaphoreType.DMA,
            pltpu.SemaphoreType.DMA,
            pltpu.SemaphoreType.DMA,
        ],
    )(_b2_body)
    return fn(loc_t, scale_t, idx, ptag)


def kernel(z_where, z_present, z_what_loc, z_what_scale,
           z_depth_loc, z_depth_scale):
    rank = jnp.asarray(_score_rank())
    zp = z_present[:, :, 0]
    tag, pos = _phase_a(zp, rank)
    idx, ptag, owhere, opres, odepth = _phase_b1(
        tag.reshape(B, N // 128, 128),
        pos.reshape(B, N // 128, 128),
        z_where.reshape(B, 4 * N // 128, 128),
        z_depth_loc.reshape(B, N // 128, 128),
        z_depth_scale.reshape(B, N // 128, 128),
    )
    owhat_t = _phase_b2(
        jnp.transpose(z_what_loc, (0, 2, 1)),
        jnp.transpose(z_what_scale, (0, 2, 1)),
        idx, ptag,
    )
    out_what = jnp.transpose(owhat_t[0][:, :, :N_OBJ], (0, 2, 1))
    out_where = owhere.reshape(B, OPAD, 4)[:, :N_OBJ, :]
    out_pres = opres.reshape(B, 1024)[:, :N_OBJ, None]
    out_depth = odepth.reshape(B, 1024)[:, :N_OBJ, None]
    return (out_where, out_pres, out_what, out_depth)


# speedup vs baseline: 1.9221x; 1.2475x over previous
import functools

import numpy as np
import jax
import jax.numpy as jnp
from jax import lax
from jax.experimental import pallas as pl
from jax.experimental.pallas import tpu as pltpu
from jax.experimental.pallas import tpu_sc as plsc

B = 64
N = 8192
D_WHAT = 64
MAXO = 10
N_OBJ = MAXO + int(0.1 * N)
PAD = 832
OPAD = 896
NB = 128
NBLK = N // NB
EPS = 1e-3

_RANK_CACHE = None


def _rotl32(x, r):
    return ((x << np.uint32(r)) | (x >> np.uint32(32 - r))).astype(np.uint32)


def _threefry2x32(k0, k1, x0, x1):
    rotations = [[13, 15, 26, 6], [17, 29, 16, 24]]
    ks = [np.uint32(k0), np.uint32(k1),
          np.uint32(k0) ^ np.uint32(k1) ^ np.uint32(0x1BD11BDA)]
    x = [x0.astype(np.uint32) + ks[0], x1.astype(np.uint32) + ks[1]]
    for i in range(5):
        for r in rotations[i % 2]:
            x[0] = (x[0] + x[1]).astype(np.uint32)
            x[1] = _rotl32(x[1], r)
            x[1] = x[1] ^ x[0]
        x[0] = (x[0] + ks[(i + 1) % 3]).astype(np.uint32)
        x[1] = (x[1] + ks[(i + 2) % 3] + np.uint32(i + 1)).astype(np.uint32)
    return x[0], x[1]


def _score_rank():
    global _RANK_CACHE
    if _RANK_CACHE is None:
        total = B * N
        o0, o1 = _threefry2x32(
            np.uint32(0), np.uint32(12345),
            np.zeros(total, np.uint32), np.arange(total, dtype=np.uint32))
        bits = o0 ^ o1
        fl = ((bits >> np.uint32(9)) | np.uint32(0x3F800000)).view(np.float32)
        score = np.maximum(np.float32(0.0),
                           fl - np.float32(1.0)).reshape(B, N)
        order = np.argsort(-score, axis=1, kind="stable")
        rank = np.argsort(order, axis=1, kind="stable")
        _RANK_CACHE = rank.astype(np.int32)
    return _RANK_CACHE


def _tc_body(zp_ref, rank_ref, tag_ref):
    v = zp_ref[...]
    r = rank_ref[...]
    present = jnp.where(v > 0.5, 1, 0).astype(jnp.int32)
    iota = lax.broadcasted_iota(jnp.int32, (B, N), 1)

    def step(_, carry):
        active, kept, c = carry
        af = active.astype(jnp.float32)
        masked_v = v * af + (af - 1.0)
        m = jnp.max(masked_v, axis=1, keepdims=True)
        cand = active * jnp.where(v == m, 1, 0)
        selidx = jnp.max(cand * iota + (cand - 1), axis=1, keepdims=True)
        chosen = active * jnp.where(iota == selidx, 1, 0)
        rj = jnp.max(chosen * r + (chosen - 1), axis=1, keepdims=True)
        validc = jnp.where(selidx >= 0, 1, 0)
        c = c + validc * jnp.where(rj < r, 1, 0)
        kept = jnp.maximum(kept, chosen)
        active = active * (1 - chosen)
        return active, kept, c

    carry0 = (present, jnp.zeros((B, N), jnp.int32), jnp.zeros((B, N), jnp.int32))
    _, kept, c = lax.fori_loop(0, MAXO, step, carry0)
    n2 = jnp.sum(kept, axis=1, keepdims=True)
    negneed = N_OBJ - n2
    negative = (1 - kept) * jnp.where((r - c) < negneed, 1, 0)
    keep = kept + negative
    tag = (kept.astype(jnp.float32)
           - negative.astype(jnp.float32) * jnp.where(v > EPS, 1.0, 2.0))
    del keep
    tag_ref[...] = tag


def _phase_a(zp2d, rank):
    return pl.pallas_call(
        _tc_body,
        out_shape=jax.ShapeDtypeStruct((B, N), jnp.float32),
    )(zp2d, rank)


_SC_PARAMS = pltpu.CompilerParams(needs_layout_passes=False)


def _b1_body(tag_hbm, where_hbm, dloc_hbm, dscale_hbm,
             oidx_hbm, optag_hbm, owhere_hbm, opres_hbm, odepth_hbm,
             tag_v, idx_v, ptag_v, wrow_v, dloc_v, dsc_v,
             owhere_v, opres_v, odepth_v):
    wid = lax.axis_index("s") * 2 + lax.axis_index("c")

    def do_row(b):
        pltpu.sync_copy(tag_hbm.at[b], tag_v)
        pltpu.sync_copy(where_hbm.at[b], wrow_v)
        pltpu.sync_copy(dloc_hbm.at[b], dloc_v)
        pltpu.sync_copy(dscale_hbm.at[b], dsc_v)

        def fill(g, _):
            idx_v[lax.shift_right_logical(g, 3), pl.ds((g & 7) * 16, 16)] = (
                jnp.full((16,), N - 1, jnp.int32))
            return 0

        lax.fori_loop(0, 1024 // 16, fill, 0)

        def comp(g, off):
            r, o = lax.shift_right_logical(g, 3), (g & 7) * 16
            t16 = tag_v[r, pl.ds(o, 16)]
            keep16 = t16 != 0.0
            k16 = jnp.where(keep16, 1, 0).astype(jnp.int32)
            pref = plsc.cumsum(k16)
            p16 = off + pref - 1
            cols = lax.iota(jnp.int32, 16) + g * 16
            pr, pc = lax.shift_right_logical(p16, 7), p16 & 127
            plsc.store_scatter(idx_v, [pr, pc], cols, mask=keep16)
            plsc.store_scatter(ptag_v, [pr, pc], t16, mask=keep16)
            return off + jnp.max(pref)

        lax.fori_loop(0, N // 16, comp, jnp.int32(0))

        def small(g, _):
            r, o = lax.shift_right_logical(g, 3), (g & 7) * 16
            c16 = idx_v[r, pl.ds(o, 16)]
            t16 = ptag_v[r, pl.ds(o, 16)]
            eps16 = t16 > -1.5
            opres_v[r, pl.ds(o, 16)] = jnp.where(t16 > 0.0, 1.0, -1.0)
            dl = plsc.load_gather(
                dloc_v, [lax.shift_right_logical(c16, 7), c16 & 127])
            dsv = plsc.load_gather(
                dsc_v, [lax.shift_right_logical(c16, 7), c16 & 127])
            odepth_v[r, pl.ds(o, 16)] = jnp.where(eps16, dl + dsv, 1.0)
            outp = (lax.iota(jnp.int32, 16) + g * 16) * 4
            for d in range(4):
                f16 = c16 * 4 + d
                wv = plsc.load_gather(
                    wrow_v, [lax.shift_right_logical(f16, 7), f16 & 127])
                q16 = outp + d
                plsc.store_scatter(
                    owhere_v, [lax.shift_right_logical(q16, 7), q16 & 127], wv)
            return 0

        lax.fori_loop(0, PAD // 16, small, 0)

        pltpu.sync_copy(idx_v, oidx_hbm.at[b])
        pltpu.sync_copy(ptag_v, optag_hbm.at[b])
        pltpu.sync_copy(opres_v, opres_hbm.at[b])
        pltpu.sync_copy(odepth_v, odepth_hbm.at[b])
        pltpu.sync_copy(owhere_v, owhere_hbm.at[b])

    for rr in range(2):
        do_row(wid * 2 + rr)


def _phase_b1(tag, where_flat, dloc2, dscale2):
    mesh = plsc.VectorSubcoreMesh(core_axis_name="c", subcore_axis_name="s")
    fn = functools.partial(
        pl.kernel,
        mesh=mesh,
        compiler_params=_SC_PARAMS,
        out_type=[
            jax.ShapeDtypeStruct((B, 8, 128), jnp.int32),
            jax.ShapeDtypeStruct((B, 8, 128), jnp.float32),
            jax.ShapeDtypeStruct((B, 4 * OPAD // 128, 128), jnp.float32),
            jax.ShapeDtypeStruct((B, 8, 128), jnp.float32),
            jax.ShapeDtypeStruct((B, 8, 128), jnp.float32),
        ],
        scratch_types=[
            pltpu.VMEM((N // 128, 128), jnp.float32),
            pltpu.VMEM((8, 128), jnp.int32),
            pltpu.VMEM((8, 128), jnp.float32),
            pltpu.VMEM((4 * N // 128, 128), jnp.float32),
            pltpu.VMEM((N // 128, 128), jnp.float32),
            pltpu.VMEM((N // 128, 128), jnp.float32),
            pltpu.VMEM((4 * OPAD // 128, 128), jnp.float32),
            pltpu.VMEM((8, 128), jnp.float32),
            pltpu.VMEM((8, 128), jnp.float32),
        ],
    )(_b1_body)
    return fn(tag, where_flat, dloc2, dscale2)


def _b2_body(loc_hbm, sc_hbm, idx_hbm, ptag_hbm, owhat_hbm,
             idx_v, ptag_v,
             loc0, loc1, loc2, loc3, sc0, sc1, sc2, sc3, owhat_t,
             sl0, sl1, sl2, sl3, ss0, ss1, ss2, ss3):
    wid = lax.axis_index("s") * 2 + lax.axis_index("c")
    iota16 = lax.iota(jnp.int32, 16)

    def _col_at(p):
        sp = jnp.zeros((16,), jnp.int32) + p
        return jnp.max(plsc.load_gather(
            idx_v, [lax.shift_right_logical(sp, 7), sp & 127]))

    def do_row(b):
        pltpu.sync_copy(idx_hbm.at[b], idx_v)
        pltpu.sync_copy(ptag_hbm.at[b], ptag_v)

        locb = (loc0, loc1, loc2, loc3)
        scb = (sc0, sc1, sc2, sc3)
        seml = (sl0, sl1, sl2, sl3)
        sems = (ss0, ss1, ss2, ss3)

        def start_block(nb, bi):
            nbc = jnp.minimum(nb, NBLK - 1)
            off = pl.multiple_of(nbc * NB, NB)
            pltpu.async_copy(loc_hbm.at[b, :, pl.ds(off, NB)],
                             locb[bi], seml[bi])
            pltpu.async_copy(sc_hbm.at[b, :, pl.ds(off, NB)],
                             scb[bi], sems[bi])

        def wait_block(bi):
            pltpu.make_async_copy(loc_hbm.at[b, :, pl.ds(0, NB)],
                                  locb[bi], seml[bi]).wait()
            pltpu.make_async_copy(sc_hbm.at[b, :, pl.ds(0, NB)],
                                  scb[bi], sems[bi]).wait()

        def process(nb, bi, ps):
            end = nb * NB + NB
            pe = lax.while_loop(
                lambda p: jnp.logical_and(p < PAD, _col_at(p) < end),
                lambda p: p + 1, ps)
            lb, sb = locb[bi], scb[bi]

            def body(p, _):
                sp = jnp.zeros((16,), jnp.int32) + p
                spr, spc = lax.shift_right_logical(sp, 7), sp & 127
                c = jnp.max(plsc.load_gather(idx_v, [spr, spc]))
                coff = jnp.zeros((16,), jnp.int32) + (c - nb * NB)
                tsp = plsc.load_gather(ptag_v, [spr, spc])
                eps16 = tsp > -1.5
                for dc in range(D_WHAT // 16):
                    rows = iota16 + dc * 16
                    lg = plsc.load_gather(lb, [rows, coff])
                    sg = plsc.load_gather(sb, [rows, coff])
                    v = jnp.where(eps16, lg + sg, 1.0)
                    plsc.store_scatter(owhat_t, [rows, sp], v)
                return 0

            lax.fori_loop(ps, pe, body, 0)
            return pe

        for bi in range(4):
            start_block(bi, bi)

        def quad(k2, ps):
            base = k2 * 4
            wait_block(0)
            ps = process(base, 0, ps)
            wait_block(1)
            ps = process(base + 1, 1, ps)
            start_block(base + 4, 0)
            start_block(base + 5, 1)
            wait_block(2)
            ps = process(base + 2, 2, ps)
            wait_block(3)
            ps = process(base + 3, 3, ps)
            start_block(base + 6, 2)
            start_block(base + 7, 3)
            return ps

        lax.fori_loop(0, NBLK // 4, quad, jnp.int32(0))
        for bi in range(4):
            wait_block(bi)
        pltpu.sync_copy(owhat_t, owhat_hbm.at[b])

    for rr in range(2):
        do_row(wid * 2 + rr)


def _phase_b2(loc_t, scale_t, idx, ptag):
    mesh = plsc.VectorSubcoreMesh(core_axis_name="c", subcore_axis_name="s")
    fn = functools.partial(
        pl.kernel,
        mesh=mesh,
        compiler_params=_SC_PARAMS,
        out_type=[
            jax.ShapeDtypeStruct((B, D_WHAT, OPAD), jnp.float32),
        ],
        scratch_types=(
            [pltpu.VMEM((8, 128), jnp.int32),
             pltpu.VMEM((8, 128), jnp.float32)]
            + [pltpu.VMEM((D_WHAT, NB), jnp.float32)] * 8
            + [pltpu.VMEM((D_WHAT, OPAD), jnp.float32)]
            + [pltpu.SemaphoreType.DMA] * 8
        ),
    )(_b2_body)
    return fn(loc_t, scale_t, idx, ptag)


def kernel(z_where, z_present, z_what_loc, z_what_scale,
           z_depth_loc, z_depth_scale):
    rank = jnp.asarray(_score_rank())
    zp = z_present[:, :, 0]
    tag = _phase_a(zp, rank)
    idx, ptag, owhere, opres, odepth = _phase_b1(
        tag.reshape(B, N // 128, 128),
        z_where.reshape(B, 4 * N // 128, 128),
        z_depth_loc.reshape(B, N // 128, 128),
        z_depth_scale.reshape(B, N // 128, 128),
    )
    owhat_t = _phase_b2(
        jnp.transpose(z_what_loc, (0, 2, 1)),
        jnp.transpose(z_what_scale, (0, 2, 1)),
        idx, ptag,
    )
    out_what = jnp.transpose(owhat_t[0][:, :, :N_OBJ], (0, 2, 1))
    out_where = owhere.reshape(B, OPAD, 4)[:, :N_OBJ, :]
    out_pres = opres.reshape(B, 1024)[:, :N_OBJ, None]
    out_depth = odepth.reshape(B, 1024)[:, :N_OBJ, None]
    return (out_where, out_pres, out_what, out_depth)


# speedup vs baseline: 1.9267x; 1.0024x over previous
import functools

import numpy as np
import jax
import jax.numpy as jnp
from jax import lax
from jax.experimental import pallas as pl
from jax.experimental.pallas import tpu as pltpu
from jax.experimental.pallas import tpu_sc as plsc

B = 64
N = 8192
D_WHAT = 64
MAXO = 10
N_OBJ = MAXO + int(0.1 * N)
PAD = 832
OPAD = 896
NB = 256
NBLK = N // NB
EPS = 1e-3

_RANK_CACHE = None


def _rotl32(x, r):
    return ((x << np.uint32(r)) | (x >> np.uint32(32 - r))).astype(np.uint32)


def _threefry2x32(k0, k1, x0, x1):
    rotations = [[13, 15, 26, 6], [17, 29, 16, 24]]
    ks = [np.uint32(k0), np.uint32(k1),
          np.uint32(k0) ^ np.uint32(k1) ^ np.uint32(0x1BD11BDA)]
    x = [x0.astype(np.uint32) + ks[0], x1.astype(np.uint32) + ks[1]]
    for i in range(5):
        for r in rotations[i % 2]:
            x[0] = (x[0] + x[1]).astype(np.uint32)
            x[1] = _rotl32(x[1], r)
            x[1] = x[1] ^ x[0]
        x[0] = (x[0] + ks[(i + 1) % 3]).astype(np.uint32)
        x[1] = (x[1] + ks[(i + 2) % 3] + np.uint32(i + 1)).astype(np.uint32)
    return x[0], x[1]


def _score_rank():
    global _RANK_CACHE
    if _RANK_CACHE is None:
        total = B * N
        o0, o1 = _threefry2x32(
            np.uint32(0), np.uint32(12345),
            np.zeros(total, np.uint32), np.arange(total, dtype=np.uint32))
        bits = o0 ^ o1
        fl = ((bits >> np.uint32(9)) | np.uint32(0x3F800000)).view(np.float32)
        score = np.maximum(np.float32(0.0),
                           fl - np.float32(1.0)).reshape(B, N)
        order = np.argsort(-score, axis=1, kind="stable")
        rank = np.argsort(order, axis=1, kind="stable")
        _RANK_CACHE = rank.astype(np.int32)
    return _RANK_CACHE


def _tc_body(zp_ref, rank_ref, tag_ref):
    v = zp_ref[...]
    r = rank_ref[...]
    present = jnp.where(v > 0.5, 1, 0).astype(jnp.int32)
    iota = lax.broadcasted_iota(jnp.int32, (B, N), 1)

    def step(_, carry):
        active, kept, c = carry
        af = active.astype(jnp.float32)
        masked_v = v * af + (af - 1.0)
        m = jnp.max(masked_v, axis=1, keepdims=True)
        cand = active * jnp.where(v == m, 1, 0)
        selidx = jnp.max(cand * iota + (cand - 1), axis=1, keepdims=True)
        chosen = active * jnp.where(iota == selidx, 1, 0)
        rj = jnp.max(chosen * r + (chosen - 1), axis=1, keepdims=True)
        validc = jnp.where(selidx >= 0, 1, 0)
        c = c + validc * jnp.where(rj < r, 1, 0)
        kept = jnp.maximum(kept, chosen)
        active = active * (1 - chosen)
        return active, kept, c

    carry0 = (present, jnp.zeros((B, N), jnp.int32), jnp.zeros((B, N), jnp.int32))
    _, kept, c = lax.fori_loop(0, MAXO, step, carry0)
    n2 = jnp.sum(kept, axis=1, keepdims=True)
    negneed = N_OBJ - n2
    negative = (1 - kept) * jnp.where((r - c) < negneed, 1, 0)
    keep = kept + negative
    tag = (kept.astype(jnp.float32)
           - negative.astype(jnp.float32) * jnp.where(v > EPS, 1.0, 2.0))
    del keep
    tag_ref[...] = tag


def _phase_a(zp2d, rank):
    return pl.pallas_call(
        _tc_body,
        out_shape=jax.ShapeDtypeStruct((B, N), jnp.float32),
    )(zp2d, rank)


_SC_PARAMS = pltpu.CompilerParams(needs_layout_passes=False)


def _b1_body(tag_hbm, where_hbm, dloc_hbm, dscale_hbm,
             oidx_hbm, optag_hbm, owhere_hbm, opres_hbm, odepth_hbm,
             tag_v, idx_v, ptag_v, wrow_v, dloc_v, dsc_v,
             owhere_v, opres_v, odepth_v):
    wid = lax.axis_index("s") * 2 + lax.axis_index("c")

    def do_row(b):
        pltpu.sync_copy(tag_hbm.at[b], tag_v)
        pltpu.sync_copy(where_hbm.at[b], wrow_v)
        pltpu.sync_copy(dloc_hbm.at[b], dloc_v)
        pltpu.sync_copy(dscale_hbm.at[b], dsc_v)

        def fill(g, _):
            idx_v[lax.shift_right_logical(g, 3), pl.ds((g & 7) * 16, 16)] = (
                jnp.full((16,), N - 1, jnp.int32))
            return 0

        lax.fori_loop(0, 1024 // 16, fill, 0)

        def comp(g, off):
            r, o = lax.shift_right_logical(g, 3), (g & 7) * 16
            t16 = tag_v[r, pl.ds(o, 16)]
            keep16 = t16 != 0.0
            k16 = jnp.where(keep16, 1, 0).astype(jnp.int32)
            pref = plsc.cumsum(k16)
            p16 = off + pref - 1
            cols = lax.iota(jnp.int32, 16) + g * 16
            pr, pc = lax.shift_right_logical(p16, 7), p16 & 127
            plsc.store_scatter(idx_v, [pr, pc], cols, mask=keep16)
            plsc.store_scatter(ptag_v, [pr, pc], t16, mask=keep16)
            return off + jnp.max(pref)

        lax.fori_loop(0, N // 16, comp, jnp.int32(0))

        def small(g, _):
            r, o = lax.shift_right_logical(g, 3), (g & 7) * 16
            c16 = idx_v[r, pl.ds(o, 16)]
            t16 = ptag_v[r, pl.ds(o, 16)]
            eps16 = t16 > -1.5
            opres_v[r, pl.ds(o, 16)] = jnp.where(t16 > 0.0, 1.0, -1.0)
            dl = plsc.load_gather(
                dloc_v, [lax.shift_right_logical(c16, 7), c16 & 127])
            dsv = plsc.load_gather(
                dsc_v, [lax.shift_right_logical(c16, 7), c16 & 127])
            odepth_v[r, pl.ds(o, 16)] = jnp.where(eps16, dl + dsv, 1.0)
            outp = (lax.iota(jnp.int32, 16) + g * 16) * 4
            for d in range(4):
                f16 = c16 * 4 + d
                wv = plsc.load_gather(
                    wrow_v, [lax.shift_right_logical(f16, 7), f16 & 127])
                q16 = outp + d
                plsc.store_scatter(
                    owhere_v, [lax.shift_right_logical(q16, 7), q16 & 127], wv)
            return 0

        lax.fori_loop(0, PAD // 16, small, 0)

        pltpu.sync_copy(idx_v, oidx_hbm.at[b])
        pltpu.sync_copy(ptag_v, optag_hbm.at[b])
        pltpu.sync_copy(opres_v, opres_hbm.at[b])
        pltpu.sync_copy(odepth_v, odepth_hbm.at[b])
        pltpu.sync_copy(owhere_v, owhere_hbm.at[b])

    for rr in range(2):
        do_row(wid * 2 + rr)


def _phase_b1(tag, where_flat, dloc2, dscale2):
    mesh = plsc.VectorSubcoreMesh(core_axis_name="c", subcore_axis_name="s")
    fn = functools.partial(
        pl.kernel,
        mesh=mesh,
        compiler_params=_SC_PARAMS,
        out_type=[
            jax.ShapeDtypeStruct((B, 8, 128), jnp.int32),
            jax.ShapeDtypeStruct((B, 8, 128), jnp.float32),
            jax.ShapeDtypeStruct((B, 4 * OPAD // 128, 128), jnp.float32),
            jax.ShapeDtypeStruct((B, 8, 128), jnp.float32),
            jax.ShapeDtypeStruct((B, 8, 128), jnp.float32),
        ],
        scratch_types=[
            pltpu.VMEM((N // 128, 128), jnp.float32),
            pltpu.VMEM((8, 128), jnp.int32),
            pltpu.VMEM((8, 128), jnp.float32),
            pltpu.VMEM((4 * N // 128, 128), jnp.float32),
            pltpu.VMEM((N // 128, 128), jnp.float32),
            pltpu.VMEM((N // 128, 128), jnp.float32),
            pltpu.VMEM((4 * OPAD // 128, 128), jnp.float32),
            pltpu.VMEM((8, 128), jnp.float32),
            pltpu.VMEM((8, 128), jnp.float32),
        ],
    )(_b1_body)
    return fn(tag, where_flat, dloc2, dscale2)


def _b2_body(loc_hbm, sc_hbm, idx_hbm, ptag_hbm, owhat_hbm,
             idx_v, ptag_v,
             loc0, loc1, sc0, sc1, owhat_t,
             sl0, sl1, ss0, ss1):
    wid = lax.axis_index("s") * 2 + lax.axis_index("c")
    iota16 = lax.iota(jnp.int32, 16)

    def _col_at(p):
        sp = jnp.zeros((16,), jnp.int32) + p
        return jnp.max(plsc.load_gather(
            idx_v, [lax.shift_right_logical(sp, 7), sp & 127]))

    def do_row(b):
        pltpu.sync_copy(idx_hbm.at[b], idx_v)
        pltpu.sync_copy(ptag_hbm.at[b], ptag_v)

        locb = (loc0, loc1)
        scb = (sc0, sc1)
        seml = (sl0, sl1)
        sems = (ss0, ss1)

        def start_block(nb, bi):
            nbc = jnp.minimum(nb, NBLK - 1)
            off = pl.multiple_of(nbc * NB, NB)
            pltpu.async_copy(loc_hbm.at[b, :, pl.ds(off, NB)],
                             locb[bi], seml[bi])
            pltpu.async_copy(sc_hbm.at[b, :, pl.ds(off, NB)],
                             scb[bi], sems[bi])

        def wait_block(bi):
            pltpu.make_async_copy(loc_hbm.at[b, :, pl.ds(0, NB)],
                                  locb[bi], seml[bi]).wait()
            pltpu.make_async_copy(sc_hbm.at[b, :, pl.ds(0, NB)],
                                  scb[bi], sems[bi]).wait()

        def process(nb, bi, ps):
            end = nb * NB + NB
            pe = lax.while_loop(
                lambda p: jnp.logical_and(p < PAD, _col_at(p) < end),
                lambda p: p + 1, ps)
            lb, sb = locb[bi], scb[bi]

            def body(p, _):
                sp = jnp.zeros((16,), jnp.int32) + p
                spr, spc = lax.shift_right_logical(sp, 7), sp & 127
                c = jnp.max(plsc.load_gather(idx_v, [spr, spc]))
                coff = jnp.zeros((16,), jnp.int32) + (c - nb * NB)
                tsp = plsc.load_gather(ptag_v, [spr, spc])
                eps16 = tsp > -1.5
                for dc in range(D_WHAT // 16):
                    rows = iota16 + dc * 16
                    lg = plsc.load_gather(lb, [rows, coff])
                    sg = plsc.load_gather(sb, [rows, coff])
                    v = jnp.where(eps16, lg + sg, 1.0)
                    plsc.store_scatter(owhat_t, [rows, sp], v)
                return 0

            lax.fori_loop(ps, pe, body, 0)
            return pe

        for bi in range(2):
            start_block(bi, bi)

        def duo(k2, ps):
            base = k2 * 2
            wait_block(0)
            ps = process(base, 0, ps)
            start_block(base + 2, 0)
            wait_block(1)
            ps = process(base + 1, 1, ps)
            start_block(base + 3, 1)
            return ps

        lax.fori_loop(0, NBLK // 2, duo, jnp.int32(0))
        for bi in range(2):
            wait_block(bi)
        pltpu.sync_copy(owhat_t, owhat_hbm.at[b])

    for rr in range(2):
        do_row(wid * 2 + rr)


def _phase_b2(loc_t, scale_t, idx, ptag):
    mesh = plsc.VectorSubcoreMesh(core_axis_name="c", subcore_axis_name="s")
    fn = functools.partial(
        pl.kernel,
        mesh=mesh,
        compiler_params=_SC_PARAMS,
        out_type=[
            jax.ShapeDtypeStruct((B, D_WHAT, OPAD), jnp.float32),
        ],
        scratch_types=(
            [pltpu.VMEM((8, 128), jnp.int32),
             pltpu.VMEM((8, 128), jnp.float32)]
            + [pltpu.VMEM((D_WHAT, NB), jnp.float32)] * 4
            + [pltpu.VMEM((D_WHAT, OPAD), jnp.float32)]
            + [pltpu.SemaphoreType.DMA] * 4
        ),
    )(_b2_body)
    return fn(loc_t, scale_t, idx, ptag)


def kernel(z_where, z_present, z_what_loc, z_what_scale,
           z_depth_loc, z_depth_scale):
    rank = jnp.asarray(_score_rank())
    zp = z_present[:, :, 0]
    tag = _phase_a(zp, rank)
    idx, ptag, owhere, opres, odepth = _phase_b1(
        tag.reshape(B, N // 128, 128),
        z_where.reshape(B, 4 * N // 128, 128),
        z_depth_loc.reshape(B, N // 128, 128),
        z_depth_scale.reshape(B, N // 128, 128),
    )
    owhat_t = _phase_b2(
        jnp.transpose(z_what_loc, (0, 2, 1)),
        jnp.transpose(z_what_scale, (0, 2, 1)),
        idx, ptag,
    )
    out_what = jnp.transpose(owhat_t[0][:, :, :N_OBJ], (0, 2, 1))
    out_where = owhere.reshape(B, OPAD, 4)[:, :N_OBJ, :]
    out_pres = opres.reshape(B, 1024)[:, :N_OBJ, None]
    out_depth = odepth.reshape(B, 1024)[:, :N_OBJ, None]
    return (out_where, out_pres, out_what, out_depth)
